# baseline (speedup 1.0000x reference)
import functools

import jax
import jax.numpy as jnp
from jax import lax
from jax.experimental import pallas as pl
from jax.experimental.pallas import tpu as pltpu
from jax.experimental.pallas import tpu_sc as plsc

B = 16384
K = 16
CHUNK = 128


@functools.partial(jax.jit, static_argnames=())
def _bpr_sc(user, pos_item, neg_item, embedding_user, embedding_item):
    info = plsc.get_sparse_core_info()
    NC, NS = info.num_cores, info.num_subcores
    NW = NC * NS
    b_per_w = B // NW
    n_chunks = b_per_w // CHUNK

    mesh = plsc.VectorSubcoreMesh(core_axis_name="c", subcore_axis_name="s")

    @functools.partial(
        pl.kernel,
        mesh=mesh,
        compiler_params=pltpu.CompilerParams(
            needs_layout_passes=False, use_tc_tiling_on_sc=False),
        out_type=jax.ShapeDtypeStruct((NW, n_chunks, CHUNK), jnp.float32),
        scratch_types=[
            pltpu.VMEM((n_chunks, CHUNK), jnp.int32),
            pltpu.VMEM((n_chunks, CHUNK), jnp.int32),
            pltpu.VMEM((n_chunks, CHUNK), jnp.int32),
            pltpu.VMEM((CHUNK, K), jnp.float32),
            pltpu.VMEM((CHUNK, K), jnp.float32),
            pltpu.VMEM((CHUNK, K), jnp.float32),
            pltpu.VMEM((CHUNK, K + 1), jnp.float32),
            pltpu.VMEM((n_chunks, CHUNK), jnp.float32),
            pltpu.SemaphoreType.DMA,
        ],
    )
    def k(user_hbm, pos_hbm, neg_hbm, eu_hbm, ei_hbm, out_hbm,
          uidx_v, pidx_v, nidx_v, u_v, p_v, n_v, d_pad, out_v, sem):
        wid = lax.axis_index("s") * NC + lax.axis_index("c")
        pltpu.sync_copy(user_hbm.at[wid], uidx_v)
        pltpu.sync_copy(pos_hbm.at[wid], pidx_v)
        pltpu.sync_copy(neg_hbm.at[wid], nidx_v)
        lane = lax.iota(jnp.int32, K)
        for c in range(n_chunks):
            pltpu.async_copy(eu_hbm.at[uidx_v.at[c]], u_v, sem).wait()
            pltpu.async_copy(ei_hbm.at[pidx_v.at[c]], p_v, sem).wait()
            pltpu.async_copy(ei_hbm.at[nidx_v.at[c]], n_v, sem).wait()

            def dbody(i, _):
                d_pad[i, pl.ds(0, K)] = u_v[i] * (p_v[i] - n_v[i])
                return 0

            lax.fori_loop(0, CHUNK, dbody, 0)

            def gbody(g, _, c=c):
                rows = g * K + lane
                acc = plsc.load_gather(d_pad, [rows, jnp.zeros((K,), jnp.int32)])
                for kk in range(1, K):
                    acc = acc + plsc.load_gather(
                        d_pad, [rows, jnp.full((K,), kk, jnp.int32)])
                out_v[c, pl.ds(g * K, K)] = acc
                return 0

            lax.fori_loop(0, CHUNK // K, gbody, 0)
        pltpu.sync_copy(out_v, out_hbm.at[wid])

    out = k(
        user.reshape(NW, n_chunks, CHUNK),
        pos_item.reshape(NW, n_chunks, CHUNK),
        neg_item.reshape(NW, n_chunks, CHUNK),
        embedding_user,
        embedding_item,
    )
    return out.reshape(B)


def kernel(user, pos_item, neg_item, embedding_user, embedding_item):
    return _bpr_sc(user, pos_item, neg_item, embedding_user, embedding_item)
